# baseline (speedup 1.0000x reference)
import math
from itertools import permutations as _permutations

import jax
import jax.numpy as jnp
import numpy as np
from jax import lax
from jax.experimental import pallas as pl
from jax.experimental.pallas import tpu as pltpu

_L = 8
_S = 4
_D = 3
_ND = 128
_ED = 16
_N = 10000
_P = 24
_PERMS = np.array(list(_permutations(range(_S))), dtype=np.int32)

_BLK = 1000
_G = _N // _BLK

_M = math.pi / 2

_ATAN_C = (0.9999999227745398, -0.3333223244657235, 0.19974024787565844,
           -0.14047793148813997, 0.10002110154691828, -0.060872867201036907,
           0.02533036269905139, -0.005020633432245819)


def _atan_pos(y):
    big = y > 1.0
    z = jnp.where(big, 1.0 / jnp.maximum(y, 1e-30), y)
    u = z * z
    p = jnp.full_like(u, _ATAN_C[-1])
    for c in _ATAN_C[-2::-1]:
        p = p * u + jnp.float32(c)
    a = z * p
    return jnp.where(big, jnp.float32(_M) - a, a)


def _intra_cols(p12):
    cur = [p12[:, 3 * s:3 * s + 3] for s in range(_S)]
    ssq = [jnp.sum(c * c, axis=-1, keepdims=True) for c in cur]
    na = [jnp.sqrt(q) for q in ssq]
    intra = []
    for s in range(_S):
        sp = (s - 1) % _S
        dot = jnp.sum(cur[sp] * cur[s], axis=-1, keepdims=True)
        intra.append(dot / jnp.maximum(na[sp] * na[s], 1e-8))
    return jnp.concatenate(intra, axis=-1), jnp.concatenate(na, axis=-1)


def _body(xn_ref, xf_ref, ed_ref, pn_ref, pf_ref, pxs_ref, ped_ref, pps_ref,
          xc_ref, out_ref, a_xn, a_xf, a_ed, a_il, a_sq):
    i = pl.program_id(0)

    @pl.when(i == 0)
    def _init():
        a_xn[...] = jnp.zeros_like(a_xn)
        a_xf[...] = jnp.zeros_like(a_xf)
        a_ed[...] = jnp.zeros_like(a_ed)
        a_il[...] = jnp.zeros_like(a_il)
        for k in range(5):
            a_sq[k] = 0.0

    xn = xn_ref[...]
    xf = xf_ref[...]
    ed = ed_ref[...]
    a_xn[...] += jnp.sum(xn, axis=0, keepdims=True)
    a_xf[...] += jnp.sum(xf, axis=0, keepdims=True)
    a_ed[...] += jnp.sum(ed, axis=0, keepdims=True)
    a_sq[0] = a_sq[0] + jnp.sum(xn * xn)
    a_sq[1] = a_sq[1] + jnp.sum(xf * xf)
    a_sq[2] = a_sq[2] + jnp.sum(ed * ed)

    pnei = pn_ref[...] - pf_ref[...]
    intra, lens = _intra_cols(pnei)
    il = jnp.concatenate([jnp.sum(intra, axis=0, keepdims=True),
                          jnp.sum(lens, axis=0, keepdims=True)], axis=-1)
    a_il[...] += il
    a_sq[3] = a_sq[3] + jnp.sum(intra * intra)
    a_sq[4] = a_sq[4] + jnp.sum(lens * lens)

    @pl.when(i == _G - 1)
    def _epilogue():
        nf = jnp.float32(_N)
        s_xn = a_xn[...]
        s_xf = a_xf[...]
        s_ed = a_ed[...]
        s_il = a_il[...]
        s_intra = s_il[:, 0:4]
        s_len = s_il[:, 4:8]
        q_xn, q_xf, q_ed, q_in, q_ln = (a_sq[0], a_sq[1], a_sq[2], a_sq[3],
                                        a_sq[4])

        iota = lax.broadcasted_iota(jnp.int32, (_P, 1), 0)
        ot = jnp.zeros((1, _L), jnp.float32)
        oi = lax.broadcasted_iota(jnp.int32, (1, _L), 1)
        for l in range(_L):
            pxs = pxs_ref[pl.ds(_P * l, _P), :]
            ped = ped_ref[pl.ds(_P * l, _P), :]
            pps = pps_ref[pl.ds(_P * l, _P), :]

            b_in, b_ln = _intra_cols(pps)
            t_ang = (q_in - 2.0 * jnp.sum(b_in * s_intra, -1, keepdims=True)
                     + nf * jnp.sum(b_in * b_in, -1, keepdims=True))
            t_len = (q_ln - 2.0 * jnp.sum(b_ln * s_len, -1, keepdims=True)
                     + nf * jnp.sum(b_ln * b_ln, -1, keepdims=True))
            t_sup = (q_xn - 2.0 * jnp.sum(pxs * s_xn, -1, keepdims=True)
                     + nf * jnp.sum(pxs * pxs, -1, keepdims=True))
            t_edg = (q_ed - 2.0 * jnp.sum(ped * s_ed, -1, keepdims=True)
                     + nf * jnp.sum(ped * ped, -1, keepdims=True))

            tmin = jnp.min(t_ang)
            bidx = jnp.min(jnp.where(t_ang <= tmin, iota, _P))
            onehot = iota == bidx
            t_len_b = jnp.sum(jnp.where(onehot, t_len, 0.0))
            t_sup_b = jnp.sum(jnp.where(onehot, t_sup, 0.0))
            t_edg_b = jnp.sum(jnp.where(onehot, t_edg, 0.0))

            xc = xc_ref[pl.ds(l, 1), :]
            t_cen = (q_xf - 2.0 * jnp.sum(xc * s_xf)
                     + nf * jnp.sum(xc * xc))

            sc_ang = _atan_pos(1.0 / tmin)
            sc_len = _atan_pos(1.0 / t_len_b)
            sc_sup = _atan_pos(1.0 / t_sup_b)
            sc_cen = _atan_pos(1.0 / t_cen)
            sc_edg = _atan_pos(1.0 / t_edg_b)

            m = jnp.float32(_M)
            tot = ((sc_len - m) ** 2 + (sc_ang - m) ** 2 + (sc_sup - m) ** 2
                   + (sc_cen - m) ** 2 + (sc_edg - m) ** 2)
            sc = _atan_pos(1.0 / tot)
            ot = ot + jnp.where(oi == l, sc, 0.0)
        out_ref[...] = ot


def _run(x_focal, p_focal, x_neighbor, p_neighbor, edge_attr_neighbor,
         x_center, x_support, edge_attr_support, p_support, interpret=False):
    n = x_focal.shape[0]
    xn2 = x_neighbor.reshape(n, _S * _ND)
    ed2 = edge_attr_neighbor.reshape(n, _S * _ED)
    pn2 = p_neighbor.reshape(n, _S * _D)
    pf4 = jnp.tile(p_focal, (1, _S))
    pxs = x_support[:, _PERMS].reshape(_L * _P, _S * _ND)
    ped = edge_attr_support[:, _PERMS].reshape(_L * _P, _S * _ED)
    pps = p_support[:, _PERMS].reshape(_L * _P, _S * _D)
    xc2 = x_center.reshape(_L, _ND)

    out = pl.pallas_call(
        _body,
        grid=(_G,),
        in_specs=[
            pl.BlockSpec((_BLK, _S * _ND), lambda i: (i, 0)),
            pl.BlockSpec((_BLK, _ND), lambda i: (i, 0)),
            pl.BlockSpec((_BLK, _S * _ED), lambda i: (i, 0)),
            pl.BlockSpec((_BLK, _S * _D), lambda i: (i, 0)),
            pl.BlockSpec((_BLK, _S * _D), lambda i: (i, 0)),
            pl.BlockSpec((_L * _P, _S * _ND), lambda i: (0, 0)),
            pl.BlockSpec((_L * _P, _S * _ED), lambda i: (0, 0)),
            pl.BlockSpec((_L * _P, _S * _D), lambda i: (0, 0)),
            pl.BlockSpec((_L, _ND), lambda i: (0, 0)),
        ],
        out_specs=pl.BlockSpec((1, _L), lambda i: (0, 0)),
        out_shape=jax.ShapeDtypeStruct((1, _L), jnp.float32),
        scratch_shapes=[
            pltpu.VMEM((1, _S * _ND), jnp.float32),
            pltpu.VMEM((1, _ND), jnp.float32),
            pltpu.VMEM((1, _S * _ED), jnp.float32),
            pltpu.VMEM((1, 2 * _S), jnp.float32),
            pltpu.SMEM((8,), jnp.float32),
        ],
        interpret=interpret,
    )(xn2, x_focal, ed2, pn2, pf4, pxs, ped, pps, xc2)
    return out.reshape(_L)


def kernel(x_focal, p_focal, x_neighbor, p_neighbor, edge_attr_neighbor,
           x_center, x_support, edge_attr_support, p_support):
    return _run(x_focal, p_focal, x_neighbor, p_neighbor, edge_attr_neighbor,
                x_center, x_support, edge_attr_support, p_support)
